# Initial kernel scaffold; baseline (speedup 1.0000x reference)
import jax
import jax.numpy as jnp
from jax.experimental import pallas as pl


def kernel(h, edge_index, source_nodes, W1_0, b1_0, g_0, be_0, W2_0, b2_0, W1_1, b1_1, g_1, be_1, W2_1, b2_1, Wi1, bi1, Wi2, bi2):
    raise NotImplementedError("write your pallas kernel here")


# speedup vs baseline: 1.8473x; 1.8473x over previous
import functools
import math

import jax
import jax.numpy as jnp
from jax import lax
from jax.experimental import pallas as pl
from jax.experimental.pallas import tpu as pltpu
from jax.experimental.pallas import tpu_sc as plsc

N = 10000
E = 160000
D = 256
TOPK = 2048

NC = 2
NS = 16
NW = NC * NS

NH = N // NC
NT = 5120
DUMP = 5100
ROWS_PER_TILE = NT // NS

EA = E // NW
EC = E // NS
CHA = 128
CHC = 64

NDP = 10016

NEG_INF = float("-inf")
I32_MIN = -2147483648

_SC_PARAMS = pltpu.CompilerParams(
    needs_layout_passes=False, use_tc_tiling_on_sc=False)


def _sc_gather_body(from_list, with_deg, *args):
    if with_deg:
        (src_hbm, dst_hbm, p_hbm, q_hbm, fr_hbm,
         x1_hbm, x2_hbm, act_hbm, deg_out,
         fbuf, snbuf, idx_s, idx_d, bufp, bufq, abuf,
         deg_sh, onesb, z16, idx8, sem1, sem2) = args
    else:
        (src_hbm, dst_hbm, p_hbm, q_hbm, fr_hbm,
         x1_hbm, x2_hbm, act_hbm,
         fbuf, snbuf, idx_s, idx_d, bufp, bufq, abuf, sem1, sem2) = args
    c = lax.axis_index("c")
    s = lax.axis_index("s")
    wid = s * NC + c
    base = wid * EA

    ones16 = jnp.full((16,), 1.0, dtype=jnp.float32)

    if from_list:
        @pl.loop(0, N // 16)
        def _zero(i):
            fbuf[pl.ds(i * 16, 16)] = jnp.zeros((16,), jnp.float32)
        pltpu.sync_copy(fr_hbm, snbuf)

        @pl.loop(0, TOPK // 16)
        def _setf(i):
            ids = snbuf[pl.ds(i * 16, 16)]
            plsc.store_scatter(fbuf, [ids], ones16)
    else:
        pltpu.sync_copy(fr_hbm, fbuf)

    if with_deg:
        @pl.loop(0, CHA)
        def _z(r):
            col1 = jnp.where(lax.iota(jnp.int32, 16) == 0, 1.0, 0.0)
            onesb[r, pl.ds(0, 16)] = col1

        @pl.loop(0, 4)
        def _z16(r):
            z16[r, pl.ds(0, 16)] = jnp.zeros((16,), jnp.float32)

        drow0 = s * (NDP // NS)

        @pl.loop(0, (NDP // NS) // 4)
        def _zdeg(i):
            pltpu.sync_copy(z16, deg_sh.at[pl.ds(drow0 + i * 4, 4)])

        plsc.subcore_barrier()

    def chunk(e0):
        pltpu.sync_copy(src_hbm.at[pl.ds(e0, CHA)], idx_s)
        pltpu.sync_copy(dst_hbm.at[pl.ds(e0, CHA)], idx_d)
        cp1 = pltpu.async_copy(p_hbm.at[idx_s], bufp, sem1)
        cp2 = pltpu.async_copy(q_hbm.at[idx_d], bufq, sem2)
        for j in range(CHA // 16):
            ids = idx_s[pl.ds(j * 16, 16)]
            abuf[pl.ds(j * 16, 16)] = plsc.load_gather(fbuf, [ids])
        cp1.wait()
        cp2.wait()
        pltpu.sync_copy(bufp, x1_hbm.at[pl.ds(e0, CHA)])
        pltpu.sync_copy(bufq, x2_hbm.at[pl.ds(e0, CHA)])
        pltpu.sync_copy(abuf, act_hbm.at[pl.ds(e0, CHA)])
        if with_deg:
            pltpu.sync_copy(onesb, deg_sh.at[idx_s], add=True)

    @pl.loop(0, EA // CHA)
    def _main(i):
        chunk(base + pl.multiple_of(i * CHA, 8))

    if EA % CHA != 0:
        tail = EA % CHA
        e0 = base + (EA // CHA) * CHA
        pltpu.sync_copy(src_hbm.at[pl.ds(base + EA - CHA, CHA)], idx_s)
        pltpu.sync_copy(dst_hbm.at[pl.ds(base + EA - CHA, CHA)], idx_d)
        cp1 = pltpu.async_copy(p_hbm.at[idx_s], bufp, sem1)
        cp2 = pltpu.async_copy(q_hbm.at[idx_d], bufq, sem2)
        for j in range(CHA // 16):
            ids = idx_s[pl.ds(j * 16, 16)]
            abuf[pl.ds(j * 16, 16)] = plsc.load_gather(fbuf, [ids])
        cp1.wait()
        cp2.wait()
        pltpu.sync_copy(bufp, x1_hbm.at[pl.ds(base + EA - CHA, CHA)])
        pltpu.sync_copy(bufq, x2_hbm.at[pl.ds(base + EA - CHA, CHA)])
        pltpu.sync_copy(abuf, act_hbm.at[pl.ds(base + EA - CHA, CHA)])
        if with_deg:
            pltpu.sync_copy(src_hbm.at[pl.ds(e0, tail)], idx8)
            pltpu.sync_copy(onesb.at[pl.ds(0, tail)], deg_sh.at[idx8], add=True)

    if with_deg:
        plsc.subcore_barrier()
        pltpu.sync_copy(deg_sh.at[pl.ds(drow0, NDP // NS)],
                        deg_out.at[c, pl.ds(drow0, NDP // NS)])


def _sc_gather(src, dst, p, q, frontier, from_list, with_deg):
    mesh = plsc.VectorSubcoreMesh(core_axis_name="c", subcore_axis_name="s")
    out_type = [
        jax.ShapeDtypeStruct((E, D), jnp.float32),
        jax.ShapeDtypeStruct((E, D), jnp.float32),
        jax.ShapeDtypeStruct((E,), jnp.float32),
    ]
    scratch = [
        pltpu.VMEM((N,), jnp.float32),
        pltpu.VMEM((TOPK,), jnp.int32),
        pltpu.VMEM((CHA,), jnp.int32),
        pltpu.VMEM((CHA,), jnp.int32),
        pltpu.VMEM((CHA, D), jnp.float32),
        pltpu.VMEM((CHA, D), jnp.float32),
        pltpu.VMEM((CHA,), jnp.float32),
    ]
    if with_deg:
        out_type.append(jax.ShapeDtypeStruct((NC, NDP, 16), jnp.float32))
        scratch += [
            pltpu.VMEM_SHARED((NDP, 16), jnp.float32),
            pltpu.VMEM((CHA, 16), jnp.float32),
            pltpu.VMEM((4, 16), jnp.float32),
            pltpu.VMEM((EA % CHA,), jnp.int32),
        ]
    scratch += [pltpu.SemaphoreType.DMA, pltpu.SemaphoreType.DMA]
    fn = pl.kernel(
        functools.partial(_sc_gather_body, from_list, with_deg),
        out_type=out_type,
        mesh=mesh,
        compiler_params=_SC_PARAMS,
        scratch_types=scratch,
    )
    return fn(src, dst, p, q, frontier)


def _sc_scatter_body(dst_hbm, act_hbm, m_hbm,
                     agg_out, cnt_out,
                     agg_sh, cnt_sh,
                     mbuf, z16, dstv, actv, idxv, idx16, onesb):
    c = lax.axis_index("c")
    s = lax.axis_index("s")
    base = s * EC
    half0 = c * NH

    @pl.loop(0, CHC)
    def _z(r):
        for j in range(D // 16):
            mbuf[r, pl.ds(j * 16, 16)] = jnp.zeros((16,), jnp.float32)
        z16[r, pl.ds(0, 16)] = jnp.zeros((16,), jnp.float32)
        col1 = jnp.where(lax.iota(jnp.int32, 16) == 0, 1.0, 0.0)
        onesb[r, pl.ds(0, 16)] = col1

    row0 = s * ROWS_PER_TILE

    @pl.loop(0, ROWS_PER_TILE // CHC)
    def _za(i):
        pltpu.sync_copy(mbuf, agg_sh.at[pl.ds(row0 + i * CHC, CHC)])
        pltpu.sync_copy(z16, cnt_sh.at[pl.ds(row0 + i * CHC, CHC)])

    plsc.subcore_barrier()

    dump = jnp.full((16,), DUMP, dtype=jnp.int32)

    def chunk(e0, sz, ibuf):
        pltpu.sync_copy(dst_hbm.at[pl.ds(e0, sz)], dstv.at[pl.ds(0, sz)])
        pltpu.sync_copy(act_hbm.at[pl.ds(e0, sz)], actv.at[pl.ds(0, sz)])
        for j in range(sz // 16):
            d = dstv[pl.ds(j * 16, 16)]
            a = actv[pl.ds(j * 16, 16)]
            loc = d - half0
            ok = (loc >= 0) & (loc < NH) & (a > 0.0)
            ibuf[pl.ds(j * 16, 16)] = jnp.where(ok, loc, dump)
        pltpu.sync_copy(m_hbm.at[pl.ds(e0, sz)], mbuf.at[pl.ds(0, sz)])
        pltpu.sync_copy(mbuf.at[pl.ds(0, sz)], agg_sh.at[ibuf], add=True)
        pltpu.sync_copy(onesb.at[pl.ds(0, sz)], cnt_sh.at[ibuf], add=True)

    @pl.loop(0, EC // CHC)
    def _main(i):
        chunk(base + pl.multiple_of(i * CHC, 8), CHC, idxv)

    if EC % CHC != 0:
        chunk(base + (EC // CHC) * CHC, EC % CHC, idx16)

    plsc.subcore_barrier()

    pltpu.sync_copy(agg_sh.at[pl.ds(row0, ROWS_PER_TILE)],
                    agg_out.at[c, pl.ds(row0, ROWS_PER_TILE)])
    pltpu.sync_copy(cnt_sh.at[pl.ds(row0, ROWS_PER_TILE)],
                    cnt_out.at[c, pl.ds(row0, ROWS_PER_TILE)])


def _sc_scatter(dst, act, m):
    mesh = plsc.VectorSubcoreMesh(core_axis_name="c", subcore_axis_name="s")
    tail = EC % CHC if EC % CHC else 16
    fn = pl.kernel(
        _sc_scatter_body,
        out_type=[
            jax.ShapeDtypeStruct((NC, NT, D), jnp.float32),
            jax.ShapeDtypeStruct((NC, NT, 16), jnp.float32),
        ],
        mesh=mesh,
        compiler_params=_SC_PARAMS,
        scratch_types=[
            pltpu.VMEM_SHARED((NT, D), jnp.float32),
            pltpu.VMEM_SHARED((NT, 16), jnp.float32),
            pltpu.VMEM((CHC, D), jnp.float32),
            pltpu.VMEM((CHC, 16), jnp.float32),
            pltpu.VMEM((CHC,), jnp.int32),
            pltpu.VMEM((CHC,), jnp.float32),
            pltpu.VMEM((CHC,), jnp.int32),
            pltpu.VMEM((tail,), jnp.int32),
            pltpu.VMEM((CHC, 16), jnp.float32),
        ],
    )
    return fn(dst, act, m)


BN = 1000
BE = 2000


def _dot(a, b):
    return jax.lax.dot_general(a, b, (((1,), (0,)), ((), ())),
                               preferred_element_type=jnp.float32)


def _tc0_body(h, wi1, bi1, wi2, bi2, w1, learned, p, q):
    hb = h[...]
    z = jnp.maximum(_dot(hb, wi1[...]) + bi1[...], 0.0)
    lr = _dot(z, wi2[...]) + bi2[...]
    learned[...] = jnp.broadcast_to(lr, (BN, 16))
    p[...] = _dot(hb, w1[0:D, :])
    q[...] = _dot(hb, w1[D:2 * D, :])


def _tc0(h, wi1, bi1, wi2, bi2, w1):
    return pl.pallas_call(
        _tc0_body,
        grid=(N // BN,),
        in_specs=[
            pl.BlockSpec((BN, D), lambda i: (i, 0)),
            pl.BlockSpec((D, D // 2), lambda i: (0, 0)),
            pl.BlockSpec((1, D // 2), lambda i: (0, 0)),
            pl.BlockSpec((D // 2, 1), lambda i: (0, 0)),
            pl.BlockSpec((1, 1), lambda i: (0, 0)),
            pl.BlockSpec((2 * D, D), lambda i: (0, 0)),
        ],
        out_specs=[
            pl.BlockSpec((BN, 16), lambda i: (i, 0)),
            pl.BlockSpec((BN, D), lambda i: (i, 0)),
            pl.BlockSpec((BN, D), lambda i: (i, 0)),
        ],
        out_shape=[
            jax.ShapeDtypeStruct((N, 16), jnp.float32),
            jax.ShapeDtypeStruct((N, D), jnp.float32),
            jax.ShapeDtypeStruct((N, D), jnp.float32),
        ],
    )(h, wi1, bi1.reshape(1, -1), wi2, bi2.reshape(1, 1), w1)


def _tcb_body(x1, x2, b1, g, be, w2, b2, m):
    x = x1[...] + x2[...] + b1[...]
    mu = jnp.mean(x, axis=-1, keepdims=True)
    xc = x - mu
    var = jnp.mean(xc * xc, axis=-1, keepdims=True)
    xn = xc * lax.rsqrt(var + 1e-5) * g[...] + be[...]
    ge = xn * 0.5 * (1.0 + lax.erf(xn * (1.0 / math.sqrt(2.0))))
    m[...] = _dot(ge, w2[...]) + b2[...]


def _tcb(x1, x2, b1, g, be, w2, b2):
    return pl.pallas_call(
        _tcb_body,
        grid=(E // BE,),
        in_specs=[
            pl.BlockSpec((BE, D), lambda i: (i, 0)),
            pl.BlockSpec((BE, D), lambda i: (i, 0)),
            pl.BlockSpec((1, D), lambda i: (0, 0)),
            pl.BlockSpec((1, D), lambda i: (0, 0)),
            pl.BlockSpec((1, D), lambda i: (0, 0)),
            pl.BlockSpec((D, D), lambda i: (0, 0)),
            pl.BlockSpec((1, D), lambda i: (0, 0)),
        ],
        out_specs=pl.BlockSpec((BE, D), lambda i: (i, 0)),
        out_shape=jax.ShapeDtypeStruct((E, D), jnp.float32),
    )(x1, x2, b1.reshape(1, -1), g.reshape(1, -1), be.reshape(1, -1),
      w2, b2.reshape(1, -1))


def _tcd1_body(h, agg, cnt, deg0, deg1, learned, w1, h1, p, q, scores):
    cr = cnt[...][:, 0:1]
    mask = (cr > 0.0).astype(jnp.float32)
    hn = h[...] + agg[...] / (cr + 1e-6) * mask
    h1[...] = hn
    p[...] = _dot(hn, w1[0:D, :])
    q[...] = _dot(hn, w1[D:2 * D, :])
    dg = deg0[...][:, 0:1] + deg1[...][:, 0:1]
    imp = 0.7 * learned[...][:, 0:1] + 0.3 * jnp.log1p(dg)
    sc = jnp.where(cr > 0.0, imp, NEG_INF)
    scores[...] = jnp.broadcast_to(sc, (BN, 16))


def _tcd1(h, agg, cnt, deg0, deg1, learned, w1):
    return pl.pallas_call(
        _tcd1_body,
        grid=(N // BN,),
        in_specs=[
            pl.BlockSpec((BN, D), lambda i: (i, 0)),
            pl.BlockSpec((BN, D), lambda i: (i, 0)),
            pl.BlockSpec((BN, 16), lambda i: (i, 0)),
            pl.BlockSpec((BN, 16), lambda i: (i, 0)),
            pl.BlockSpec((BN, 16), lambda i: (i, 0)),
            pl.BlockSpec((BN, 16), lambda i: (i, 0)),
            pl.BlockSpec((2 * D, D), lambda i: (0, 0)),
        ],
        out_specs=[
            pl.BlockSpec((BN, D), lambda i: (i, 0)),
            pl.BlockSpec((BN, D), lambda i: (i, 0)),
            pl.BlockSpec((BN, D), lambda i: (i, 0)),
            pl.BlockSpec((BN, 16), lambda i: (i, 0)),
        ],
        out_shape=[
            jax.ShapeDtypeStruct((N, D), jnp.float32),
            jax.ShapeDtypeStruct((N, D), jnp.float32),
            jax.ShapeDtypeStruct((N, D), jnp.float32),
            jax.ShapeDtypeStruct((N, 16), jnp.float32),
        ],
    )(h, agg, cnt, deg0, deg1, learned, w1)


def _tcdf_body(h, agg, cnt, h1):
    cr = cnt[...][:, 0:1]
    mask = (cr > 0.0).astype(jnp.float32)
    h1[...] = h[...] + agg[...] / (cr + 1e-6) * mask


def _tcdf(h, agg, cnt):
    return pl.pallas_call(
        _tcdf_body,
        grid=(N // BN,),
        in_specs=[
            pl.BlockSpec((BN, D), lambda i: (i, 0)),
            pl.BlockSpec((BN, D), lambda i: (i, 0)),
            pl.BlockSpec((BN, 16), lambda i: (i, 0)),
        ],
        out_specs=pl.BlockSpec((BN, D), lambda i: (i, 0)),
        out_shape=jax.ShapeDtypeStruct((N, D), jnp.float32),
    )(h, agg, cnt)


NPAD = 10240


def _topk_body(sref, out):
    s = sref[...]
    b = lax.bitcast_convert_type(s, jnp.int32)
    k = b ^ lax.shift_right_arithmetic(b, 31) & jnp.int32(0x7FFFFFFF)

    def body(i, t):
        bit = lax.shift_left(jnp.int32(1), 31 - i)
        t2 = t | bit
        thr = t2 ^ I32_MIN
        c = jnp.sum((k >= thr).astype(jnp.int32))
        return jnp.where(c >= TOPK, t2, t)

    tu = lax.fori_loop(0, 32, body, jnp.int32(0))
    tk = tu ^ I32_MIN
    n_gt = jnp.sum((k > tk).astype(jnp.int32))
    r = (TOPK - n_gt).astype(jnp.float32)
    eq = (k == tk).astype(jnp.float32)
    cols = NPAD // 80
    um = (lax.broadcasted_iota(jnp.int32, (cols, cols), 0)
          < lax.broadcasted_iota(jnp.int32, (cols, cols), 1)).astype(jnp.float32)
    rowpre = _dot(eq, um)
    rowsum = jnp.sum(eq, axis=1, keepdims=True)
    lt = (lax.broadcasted_iota(jnp.int32, (80, 80), 1)
          < lax.broadcasted_iota(jnp.int32, (80, 80), 0)).astype(jnp.float32)
    rowoff = _dot(lt, rowsum)
    rank = rowpre + rowoff
    ind = (k > tk) | ((k == tk) & (rank < r))
    out[...] = ind.astype(jnp.float32)


def _topk(scores_col):
    pad = jnp.full((NPAD - N,), NEG_INF, dtype=jnp.float32)
    s2 = jnp.concatenate([scores_col, pad]).reshape(80, NPAD // 80)
    ind = pl.pallas_call(
        _topk_body,
        grid=(1,),
        in_specs=[pl.BlockSpec((80, NPAD // 80), lambda i: (0, 0))],
        out_specs=pl.BlockSpec((80, NPAD // 80), lambda i: (0, 0)),
        out_shape=jax.ShapeDtypeStruct((80, NPAD // 80), jnp.float32),
    )(s2)
    return ind.reshape(NPAD)[:N]


def kernel(h, edge_index, source_nodes,
           W1_0, b1_0, g_0, be_0, W2_0, b2_0,
           W1_1, b1_1, g_1, be_1, W2_1, b2_1,
           Wi1, bi1, Wi2, bi2):
    src = edge_index[0]
    dst = edge_index[1]

    learned, p0, q0 = _tc0(h, Wi1, bi1, Wi2, bi2, W1_0)

    x1, x2, act, degh = _sc_gather(src, dst, p0, q0, source_nodes,
                                   from_list=True, with_deg=True)
    m = _tcb(x1, x2, b1_0, g_0, be_0, W2_0, b2_0)
    aggh, cnth = _sc_scatter(dst, act, m)
    agg = jnp.concatenate([aggh[0, :NH], aggh[1, :NH]], axis=0)
    cnt = jnp.concatenate([cnth[0, :NH], cnth[1, :NH]], axis=0)
    h1, p1, q1, scores = _tcd1(h, agg, cnt, degh[0, :N], degh[1, :N],
                               learned, W1_1)
    frontier = _topk(scores[:, 0])

    x1b, x2b, actb = _sc_gather(src, dst, p1, q1, frontier,
                                from_list=False, with_deg=False)
    m2 = _tcb(x1b, x2b, b1_1, g_1, be_1, W2_1, b2_1)
    aggh2, cnth2 = _sc_scatter(dst, actb, m2)
    agg2 = jnp.concatenate([aggh2[0, :NH], aggh2[1, :NH]], axis=0)
    cnt2 = jnp.concatenate([cnth2[0, :NH], cnth2[1, :NH]], axis=0)
    return _tcdf(h1, agg2, cnt2)
